# baseline (device time: 11067 ns/iter reference)
import jax
import jax.numpy as jnp
from jax import lax
from jax.experimental import pallas as pl
from jax.experimental.pallas import tpu as pltpu

K = 8
NEG = float(jnp.finfo(jnp.float32).min)
ROWS = 256
RB = ROWS // 4
NDEV = 8

_OFFSETS = [
    (ox, oy, oz)
    for ox in (0, 1)
    for oy in (0, 1)
    for oz in (0, 1)
    if (ox, oy, oz) != (0, 0, 0)
]


def _topk_desc(v, k):
    cols = []
    for _ in range(k):
        mx = jnp.max(v, axis=1, keepdims=True)
        cols.append(mx)
        v = jnp.where(v == mx, NEG, v)
    return jnp.concatenate(cols, axis=1)


def kernel(x):
    m, n = x.shape

    def body(x_ref, out_ref, cand_ref, peers_ref, send_sems, recv_sems):
        my_x = lax.axis_index("x")
        my_y = lax.axis_index("y")
        my_z = lax.axis_index("z")
        my_b = 2 * my_x + my_y
        my_slot = 2 * my_b + my_z

        def dest(ox, oy, oz):
            return ((my_x + ox) % 2, (my_y + oy) % 2, (my_z + oz) % 2)

        barrier_sem = pltpu.get_barrier_semaphore()
        for off in _OFFSETS:
            pl.semaphore_signal(
                barrier_sem,
                inc=1,
                device_id=dest(*off),
                device_id_type=pl.DeviceIdType.MESH,
            )
        pl.semaphore_wait(barrier_sem, len(_OFFSETS))

        for b in range(4):

            @pl.when(my_b == b)
            def _():
                cand_ref[:, :] = _topk_desc(x_ref[b * RB : (b + 1) * RB, :], K)

        rdmas = []
        for i, off in enumerate(_OFFSETS):
            rdma = pltpu.make_async_remote_copy(
                src_ref=cand_ref,
                dst_ref=peers_ref.at[my_slot],
                send_sem=send_sems.at[i],
                recv_sem=recv_sems.at[my_slot],
                device_id=dest(*off),
                device_id_type=pl.DeviceIdType.MESH,
            )
            rdma.start()
            rdmas.append(rdma)

        for s in range(NDEV):

            @pl.when(my_slot == s)
            def _():
                peers_ref[s, :, :] = cand_ref[:, :]

        for s in range(NDEV):

            @pl.when(my_slot != s)
            def _():
                pltpu.make_async_remote_copy(
                    src_ref=cand_ref,
                    dst_ref=peers_ref.at[s],
                    send_sem=send_sems.at[0],
                    recv_sem=recv_sems.at[s],
                    device_id=dest(0, 0, 1),
                    device_id_type=pl.DeviceIdType.MESH,
                ).wait_recv()

        both = jnp.concatenate(
            [
                jnp.concatenate(
                    [peers_ref[2 * b, :, :], peers_ref[2 * b + 1, :, :]], axis=1
                )
                for b in range(4)
            ],
            axis=0,
        )
        out_ref[:, :] = _topk_desc(both, K)

        for rdma in rdmas:
            rdma.wait_send()

    return pl.pallas_call(
        body,
        out_shape=jax.ShapeDtypeStruct((ROWS, K), jnp.float32),
        in_specs=[pl.BlockSpec(memory_space=pltpu.VMEM)],
        out_specs=pl.BlockSpec(memory_space=pltpu.VMEM),
        scratch_shapes=[
            pltpu.VMEM((RB, K), jnp.float32),
            pltpu.VMEM((NDEV, RB, K), jnp.float32),
            pltpu.SemaphoreType.DMA((len(_OFFSETS),)),
            pltpu.SemaphoreType.DMA((NDEV,)),
        ],
        compiler_params=pltpu.CompilerParams(collective_id=0),
    )(x)


# device time: 8814 ns/iter; 1.2556x vs baseline; 1.2556x over previous
import jax
import jax.numpy as jnp
from jax import lax
from jax.experimental import pallas as pl
from jax.experimental.pallas import tpu as pltpu

K = 8
NEG = float(jnp.finfo(jnp.float32).min)


def _topk_desc(v, k):
    cols = []
    for _ in range(k):
        mx = jnp.max(v, axis=1, keepdims=True)
        cols.append(mx)
        v = jnp.where(v == mx, NEG, v)
    return jnp.concatenate(cols, axis=1)


def kernel(x):
    m, n = x.shape

    def body(x_ref, out_ref, cand_ref, peer_ref, send_sem, recv_sem):
        my_x = lax.axis_index("x")
        my_y = lax.axis_index("y")
        my_z = lax.axis_index("z")
        peer = (my_x, my_y, 1 - my_z)

        barrier_sem = pltpu.get_barrier_semaphore()
        pl.semaphore_signal(
            barrier_sem,
            inc=1,
            device_id=peer,
            device_id_type=pl.DeviceIdType.MESH,
        )

        mine = _topk_desc(x_ref[:, :], K)
        cand_ref[:, :] = mine

        pl.semaphore_wait(barrier_sem, 1)

        rdma = pltpu.make_async_remote_copy(
            src_ref=cand_ref,
            dst_ref=peer_ref,
            send_sem=send_sem,
            recv_sem=recv_sem,
            device_id=peer,
            device_id_type=pl.DeviceIdType.MESH,
        )
        rdma.start()
        rdma.wait_recv()

        both = jnp.concatenate([mine, peer_ref[:, :]], axis=1)
        out_ref[:, :] = _topk_desc(both, K)

        rdma.wait_send()

    return pl.pallas_call(
        body,
        out_shape=jax.ShapeDtypeStruct((m, K), jnp.float32),
        in_specs=[pl.BlockSpec(memory_space=pltpu.VMEM)],
        out_specs=pl.BlockSpec(memory_space=pltpu.VMEM),
        scratch_shapes=[
            pltpu.VMEM((m, K), jnp.float32),
            pltpu.VMEM((m, K), jnp.float32),
            pltpu.SemaphoreType.DMA,
            pltpu.SemaphoreType.DMA,
        ],
        compiler_params=pltpu.CompilerParams(collective_id=0),
    )(x)
